# baseline (speedup 1.0000x reference)
import functools

import jax
import jax.numpy as jnp
from jax import lax
from jax.experimental import pallas as pl
from jax.experimental.pallas import tpu as pltpu
from jax.experimental.pallas import tpu_sc as plsc

TOKENS = 8192
D_MODEL = 1024
N_EXPERTS = 16
NW = 32
NCH = D_MODEL // 16

T_SC = 512
T_TC = TOKENS - T_SC
BT = 1920


def _sc_gate(t0, n_tok):
    tpw = n_tok // NW
    cht = min(32, tpw)
    ngrp = cht // 16
    nst = tpw // cht
    mesh = plsc.VectorSubcoreMesh(core_axis_name="c", subcore_axis_name="s")

    @functools.partial(
        pl.kernel,
        out_type=jax.ShapeDtypeStruct((n_tok, N_EXPERTS), jnp.float32),
        mesh=mesh,
        compiler_params=pltpu.CompilerParams(needs_layout_passes=False),
        scratch_types=[
            pltpu.MemorySpace.VMEM((cht, D_MODEL), jnp.float32),
            pltpu.MemorySpace.VMEM((N_EXPERTS, D_MODEL), jnp.float32),
            pltpu.MemorySpace.VMEM((cht, N_EXPERTS), jnp.float32),
            pltpu.MemorySpace.VMEM((16, 16), jnp.float32),
            pltpu.MemorySpace.VMEM((16, 16), jnp.float32),
        ],
    )
    def body(x_hbm, wt_hbm, o_hbm, xv, wtv, ov, prv, egv):
        wid = lax.axis_index("s") * 2 + lax.axis_index("c")
        pltpu.sync_copy(wt_hbm, wtv)
        lane = lax.iota(jnp.int32, 16)

        def stage(st, _):
            off = wid * tpw + st * cht
            pltpu.sync_copy(x_hbm.at[pl.ds(t0 + off, cht), :], xv)

            def group(g, _):
                def tok(tl, _):
                    t = g * 16 + tl

                    def chunk(c, accs):
                        xc = xv[t, pl.ds(c * 16, 16)]
                        return tuple(
                            accs[e] + xc * wtv[e, pl.ds(c * 16, 16)]
                            for e in range(N_EXPERTS)
                        )

                    accs = lax.fori_loop(
                        0, NCH, chunk,
                        tuple(jnp.zeros((16,), jnp.float32)
                              for _ in range(N_EXPERTS)),
                    )
                    for e in range(N_EXPERTS):
                        prv[e] = accs[e]
                    logits = plsc.load_gather(
                        prv, [lane, jnp.full((16,), 0, jnp.int32)])
                    for l in range(1, 16):
                        logits = logits + plsc.load_gather(
                            prv, [lane, jnp.full((16,), l, jnp.int32)])
                    egv[tl] = jnp.exp(logits)
                    return 0

                lax.fori_loop(0, 16, tok, 0)
                cols = [
                    plsc.load_gather(egv, [lane, jnp.full((16,), e, jnp.int32)])
                    for e in range(N_EXPERTS)
                ]
                s = cols[0]
                for e in range(1, N_EXPERTS):
                    s = s + cols[e]
                r = 1.0 / s
                for e in range(N_EXPERTS):
                    plsc.store_scatter(
                        ov,
                        [g * 16 + lane, jnp.full((16,), e, jnp.int32)],
                        cols[e] * r,
                    )
                return 0

            lax.fori_loop(0, ngrp, group, 0)
            pltpu.sync_copy(ov, o_hbm.at[pl.ds(off, cht), :])
            return 0

        lax.fori_loop(0, nst, stage, 0)

    return body


def _tc_block(x_ref, w_ref, o_ref):
    xb = x_ref[...].astype(jnp.bfloat16)
    wb = w_ref[...].astype(jnp.bfloat16)
    logits = jnp.dot(xb, wb, preferred_element_type=jnp.float32)
    m = jnp.max(logits, axis=-1, keepdims=True)
    e = jnp.exp(logits - m)
    o_ref[...] = e / jnp.sum(e, axis=-1, keepdims=True)


def _tc_gate(x, W):
    return pl.pallas_call(
        _tc_block,
        grid=(T_TC // BT,),
        in_specs=[
            pl.BlockSpec((BT, D_MODEL), lambda i: (i, 0)),
            pl.BlockSpec((D_MODEL, N_EXPERTS), lambda i: (0, 0)),
        ],
        out_specs=pl.BlockSpec((BT, N_EXPERTS), lambda i: (i, 0)),
        out_shape=jax.ShapeDtypeStruct((T_TC, N_EXPERTS), jnp.float32),
        compiler_params=pltpu.CompilerParams(
            dimension_semantics=("parallel",)
        ),
    )(x, W)


def kernel(x, W):
    wt = W.T
    sc_out = _sc_gate(T_TC, T_SC)(x, wt)
    tc_out = _tc_gate(x, W)
    return tc_out, sc_out


# speedup vs baseline: 2.1484x; 2.1484x over previous
import jax
import jax.numpy as jnp
from jax.experimental import pallas as pl
from jax.experimental.pallas import tpu as pltpu

TOKENS = 8192
D_MODEL = 1024
N_EXPERTS = 16
BT = 2048


def _gate_block(x_ref, w_ref, o_ref):
    xb = x_ref[...].astype(jnp.bfloat16)
    wb = w_ref[...].astype(jnp.bfloat16)
    logits = jnp.dot(xb, wb, preferred_element_type=jnp.float32)
    m = jnp.max(logits, axis=-1, keepdims=True)
    e = jnp.exp(logits - m)
    o_ref[...] = e / jnp.sum(e, axis=-1, keepdims=True)


def kernel(x, W):
    return pl.pallas_call(
        _gate_block,
        grid=(TOKENS // BT,),
        in_specs=[
            pl.BlockSpec((BT, D_MODEL), lambda i: (i, 0)),
            pl.BlockSpec((D_MODEL, N_EXPERTS), lambda i: (0, 0)),
        ],
        out_specs=pl.BlockSpec((BT, N_EXPERTS), lambda i: (i, 0)),
        out_shape=jax.ShapeDtypeStruct((TOKENS, N_EXPERTS), jnp.float32),
        compiler_params=pltpu.CompilerParams(
            dimension_semantics=("parallel",),
            skip_device_barrier=True,
        ),
    )(x, W)


# speedup vs baseline: 2.1546x; 1.0029x over previous
import jax
import jax.numpy as jnp
from jax.experimental import pallas as pl
from jax.experimental.pallas import tpu as pltpu

TOKENS = 8192
D_MODEL = 1024
N_EXPERTS = 16
BT = 2048


def _gate_block(x_ref, w_ref, o_ref):
    xb = x_ref[...].astype(jnp.bfloat16)
    wb = w_ref[...].astype(jnp.bfloat16)
    logits = jnp.dot(xb, wb, preferred_element_type=jnp.float32)
    m = jnp.max(logits, axis=-1, keepdims=True)
    e = jnp.exp(logits - m)
    o_ref[...] = e / jnp.sum(e, axis=-1, keepdims=True)


def kernel(x, W):
    return pl.pallas_call(
        _gate_block,
        grid=(TOKENS // BT,),
        in_specs=[
            pl.BlockSpec((BT, D_MODEL), lambda i: (i, 0)),
            pl.BlockSpec((D_MODEL, N_EXPERTS), lambda i: (0, 0)),
        ],
        out_specs=pl.BlockSpec((BT, N_EXPERTS), lambda i: (i, 0)),
        out_shape=jax.ShapeDtypeStruct((TOKENS, N_EXPERTS), jnp.float32),
        compiler_params=pltpu.CompilerParams(
            dimension_semantics=("parallel",)
        ),
    )(x, W)
